# baseline (speedup 1.0000x reference)
import jax
import jax.numpy as jnp
from jax import lax
from jax.experimental import pallas as pl
from jax.experimental.pallas import tpu as pltpu
from jax.experimental.pallas import tpu_sc as plsc

BATCH = 16384
N_FIELDS = 26
VOCAB = 100000
EMBED_DIM = 32
CONT_DIM = 13
OUT_DIM = CONT_DIM + N_FIELDS * EMBED_DIM
EMB_W = N_FIELDS * EMBED_DIM

NC = 2
NS = 16
LANES = 16
NW = NC * NS
ROWS_W = BATCH // NW
CR = 32
NCHUNK = ROWS_W // CR
IDX_N = CR * N_FIELDS
GRPS = IDX_N // LANES


def _body(cont_hbm, cat_hbm, tab_hbm, out_hbm,
          cat_v, cont_v, idx_v, rows_v, out_v, sem_g):
    wid = lax.axis_index("s") * NC + lax.axis_index("c")
    base = wid * ROWS_W

    lanes = lax.iota(jnp.int32, LANES)

    def chunk(c, _):
        row0 = base + c * CR
        pltpu.sync_copy(cat_hbm.at[pl.ds(row0 * N_FIELDS, IDX_N)], cat_v)
        pltpu.sync_copy(cont_hbm.at[pl.ds(row0 * CONT_DIM, CR * CONT_DIM)],
                        cont_v.at[pl.ds(0, CR * CONT_DIM)])

        def build(g, _):
            p = g * LANES + lanes
            f = lax.rem(p, N_FIELDS)
            idx_v[pl.ds(g * LANES, LANES)] = (
                cat_v[pl.ds(g * LANES, LANES)] + f * VOCAB)
            return 0

        lax.fori_loop(0, GRPS, build, 0)

        pltpu.async_copy(tab_hbm.at[idx_v], rows_v, sem_g).wait()

        def asm_row(i, _):
            out_v[pl.ds(i * OUT_DIM, LANES)] = cont_v[pl.ds(i * CONT_DIM, LANES)]

            def asm_j(j, _):
                r = i * N_FIELDS + j
                d = i * OUT_DIM + CONT_DIM + j * EMBED_DIM
                out_v[pl.ds(d, LANES)] = rows_v[r, pl.ds(0, LANES)]
                out_v[pl.ds(d + LANES, LANES)] = rows_v[r, pl.ds(LANES, LANES)]
                return 0

            return lax.fori_loop(0, N_FIELDS, asm_j, 0)

        lax.fori_loop(0, CR, asm_row, 0)

        pltpu.sync_copy(out_v, out_hbm.at[pl.ds(row0 * OUT_DIM, CR * OUT_DIM)])
        return 0

    lax.fori_loop(0, NCHUNK, chunk, 0)


@jax.jit
def kernel(continuous_data, categorical_data, tables):
    cat = categorical_data.astype(jnp.int32).reshape(BATCH * N_FIELDS)
    cont = continuous_data.reshape(BATCH * CONT_DIM)
    tab = tables.reshape(N_FIELDS * VOCAB, EMBED_DIM)
    mesh = plsc.VectorSubcoreMesh(core_axis_name="c", subcore_axis_name="s")
    run = pl.kernel(
        _body,
        out_type=jax.ShapeDtypeStruct((BATCH * OUT_DIM,), jnp.float32),
        mesh=mesh,
        scratch_types=[
            pltpu.VMEM((IDX_N,), jnp.int32),
            pltpu.VMEM((CR * CONT_DIM + LANES,), jnp.float32),
            pltpu.VMEM((IDX_N,), jnp.int32),
            pltpu.VMEM((IDX_N, EMBED_DIM), jnp.float32),
            pltpu.VMEM((CR * OUT_DIM,), jnp.float32),
            pltpu.SemaphoreType.DMA,
        ],
        compiler_params=pltpu.CompilerParams(use_tc_tiling_on_sc=False),
    )
    out = run(cont, cat, tab)
    return out.reshape(BATCH, OUT_DIM)


# speedup vs baseline: 1.0411x; 1.0411x over previous
import jax
import jax.numpy as jnp
from jax import lax
from jax.experimental import pallas as pl
from jax.experimental.pallas import tpu as pltpu
from jax.experimental.pallas import tpu_sc as plsc

BATCH = 16384
N_FIELDS = 26
VOCAB = 100000
EMBED_DIM = 32
CONT_DIM = 13
OUT_DIM = CONT_DIM + N_FIELDS * EMBED_DIM
EMB_W = N_FIELDS * EMBED_DIM

NC = 2
NS = 16
LANES = 16
NW = NC * NS
ROWS_W = BATCH // NW
CR = 32
NCHUNK = ROWS_W // CR
IDX_N = CR * N_FIELDS
GRPS = IDX_N // LANES


def _body(cont_hbm, cat_hbm, tab_hbm, out_hbm,
          cat_v, cont_v, idx_v, rows_v, out_v,
          sem_cat, sem_cont, sem_g, sem_w):
    wid = lax.axis_index("s") * NC + lax.axis_index("c")
    base = wid * ROWS_W

    lanes = lax.iota(jnp.int32, LANES)

    def start_in(c):
        b = c % 3
        row0 = base + c * CR
        pltpu.async_copy(cat_hbm.at[pl.ds(row0 * N_FIELDS, IDX_N)],
                         cat_v.at[b], sem_cat.at[b])
        pltpu.async_copy(cont_hbm.at[pl.ds(row0 * CONT_DIM, CR * CONT_DIM)],
                         cont_v.at[b, pl.ds(0, CR * CONT_DIM)],
                         sem_cont.at[b])

    def wait_in(c):
        b = c % 3
        row0 = base + c * CR
        pltpu.make_async_copy(cat_hbm.at[pl.ds(row0 * N_FIELDS, IDX_N)],
                              cat_v.at[b], sem_cat.at[b]).wait()
        pltpu.make_async_copy(cont_hbm.at[pl.ds(row0 * CONT_DIM, CR * CONT_DIM)],
                              cont_v.at[b, pl.ds(0, CR * CONT_DIM)],
                              sem_cont.at[b]).wait()

    def build_idx(c):
        b3, b2 = c % 3, c & 1

        def build(g, _):
            p = g * LANES + lanes
            f = lax.rem(p, N_FIELDS)
            idx_v[b2, pl.ds(g * LANES, LANES)] = (
                cat_v[b3, pl.ds(g * LANES, LANES)] + f * VOCAB)
            return 0

        lax.fori_loop(0, GRPS, build, 0)
        pltpu.async_copy(tab_hbm.at[idx_v.at[b2]], rows_v.at[b2], sem_g.at[b2])

    def wait_gather(c):
        b2 = c & 1
        pltpu.make_async_copy(tab_hbm.at[idx_v.at[b2]], rows_v.at[b2],
                              sem_g.at[b2]).wait()

    def assemble(c):
        b3, b2 = c % 3, c & 1

        def asm_row(i, _):
            out_v[b2, pl.ds(i * OUT_DIM, LANES)] = (
                cont_v[b3, pl.ds(i * CONT_DIM, LANES)])

            def asm_j(j, _):
                r = i * N_FIELDS + j
                d = i * OUT_DIM + CONT_DIM + j * EMBED_DIM
                out_v[b2, pl.ds(d, LANES)] = rows_v[b2, r, pl.ds(0, LANES)]
                out_v[b2, pl.ds(d + LANES, LANES)] = (
                    rows_v[b2, r, pl.ds(LANES, LANES)])
                return 0

            return lax.fori_loop(0, N_FIELDS, asm_j, 0)

        lax.fori_loop(0, CR, asm_row, 0)

    def out_slice(c):
        row0 = base + c * CR
        return out_hbm.at[pl.ds(row0 * OUT_DIM, CR * OUT_DIM)]

    def start_write(c):
        pltpu.async_copy(out_v.at[c & 1], out_slice(c), sem_w.at[c & 1])

    def wait_write(c):
        pltpu.make_async_copy(out_v.at[c & 1], out_slice(c),
                              sem_w.at[c & 1]).wait()

    start_in(0)
    start_in(1)
    for c in range(NCHUNK):
        wait_in(c)
        build_idx(c)
        if c >= 1:
            wait_gather(c - 1)
            if c >= 3:
                wait_write(c - 3)
            assemble(c - 1)
            start_write(c - 1)
        if c + 2 < NCHUNK:
            start_in(c + 2)

    wait_gather(NCHUNK - 1)
    wait_write(NCHUNK - 3)
    assemble(NCHUNK - 1)
    start_write(NCHUNK - 1)
    wait_write(NCHUNK - 2)
    wait_write(NCHUNK - 1)


@jax.jit
def kernel(continuous_data, categorical_data, tables):
    cat = categorical_data.astype(jnp.int32).reshape(BATCH * N_FIELDS)
    cont = continuous_data.reshape(BATCH * CONT_DIM)
    tab = tables.reshape(N_FIELDS * VOCAB, EMBED_DIM)
    mesh = plsc.VectorSubcoreMesh(core_axis_name="c", subcore_axis_name="s")
    run = pl.kernel(
        _body,
        out_type=jax.ShapeDtypeStruct((BATCH * OUT_DIM,), jnp.float32),
        mesh=mesh,
        scratch_types=[
            pltpu.VMEM((3, IDX_N,), jnp.int32),
            pltpu.VMEM((3, CR * CONT_DIM + LANES,), jnp.float32),
            pltpu.VMEM((2, IDX_N,), jnp.int32),
            pltpu.VMEM((2, IDX_N, EMBED_DIM), jnp.float32),
            pltpu.VMEM((2, CR * OUT_DIM,), jnp.float32),
            pltpu.SemaphoreType.DMA((3,)),
            pltpu.SemaphoreType.DMA((3,)),
            pltpu.SemaphoreType.DMA((2,)),
            pltpu.SemaphoreType.DMA((2,)),
        ],
        compiler_params=pltpu.CompilerParams(use_tc_tiling_on_sc=False),
    )
    out = run(cont, cat, tab)
    return out.reshape(BATCH, OUT_DIM)


# speedup vs baseline: 1.0597x; 1.0178x over previous
import jax
import jax.numpy as jnp
from jax import lax
from jax.experimental import pallas as pl
from jax.experimental.pallas import tpu as pltpu
from jax.experimental.pallas import tpu_sc as plsc

BATCH = 16384
N_FIELDS = 26
VOCAB = 100000
EMBED_DIM = 32
CONT_DIM = 13
OUT_DIM = CONT_DIM + N_FIELDS * EMBED_DIM

NC = 2
NS = 16
LANES = 16
NW = NC * NS
ROWS_W = BATCH // NW
CR = 32
NCHUNK = ROWS_W // CR


def _body(cont_hbm, catt_hbm, tab_hbm, out_hbm,
          catw_v, cont_v, rows_v, out_v,
          sem_cat, sem_cont, sem_g, sem_w):
    wid = lax.axis_index("s") * NC + lax.axis_index("c")
    base = wid * ROWS_W

    pltpu.sync_copy(catt_hbm.at[:, pl.ds(base, ROWS_W)], catw_v)

    def start_in(c):
        b = c % 3
        row0 = base + c * CR
        pltpu.async_copy(cont_hbm.at[pl.ds(row0 * CONT_DIM, CR * CONT_DIM)],
                         cont_v.at[b, pl.ds(0, CR * CONT_DIM)],
                         sem_cont.at[b])

    def wait_in(c):
        b = c % 3
        row0 = base + c * CR
        pltpu.make_async_copy(cont_hbm.at[pl.ds(row0 * CONT_DIM, CR * CONT_DIM)],
                              cont_v.at[b, pl.ds(0, CR * CONT_DIM)],
                              sem_cont.at[b]).wait()

    def start_gathers(c):
        b2 = c & 1

        def fire(f, _):
            pltpu.async_copy(
                tab_hbm.at[f].at[catw_v.at[f, pl.ds(c * CR, CR)]],
                rows_v.at[b2, f], sem_g.at[b2])
            return 0

        lax.fori_loop(0, N_FIELDS, fire, 0)

    def wait_gathers(c):
        b2 = c & 1

        def drain(f, _):
            pltpu.make_async_copy(
                tab_hbm.at[f].at[catw_v.at[f, pl.ds(c * CR, CR)]],
                rows_v.at[b2, f], sem_g.at[b2]).wait()
            return 0

        lax.fori_loop(0, N_FIELDS, drain, 0)

    def assemble(c):
        b3, b2 = c % 3, c & 1

        def asm_row(i, _):
            out_v[b2, pl.ds(i * OUT_DIM, LANES)] = (
                cont_v[b3, pl.ds(i * CONT_DIM, LANES)])

            def asm_j(j, _):
                d = i * OUT_DIM + CONT_DIM + j * EMBED_DIM
                out_v[b2, pl.ds(d, LANES)] = rows_v[b2, j, i, pl.ds(0, LANES)]
                out_v[b2, pl.ds(d + LANES, LANES)] = (
                    rows_v[b2, j, i, pl.ds(LANES, LANES)])
                return 0

            return lax.fori_loop(0, N_FIELDS, asm_j, 0)

        lax.fori_loop(0, CR, asm_row, 0)

    def out_slice(c):
        row0 = base + c * CR
        return out_hbm.at[pl.ds(row0 * OUT_DIM, CR * OUT_DIM)]

    def start_write(c):
        pltpu.async_copy(out_v.at[c & 1], out_slice(c), sem_w.at[c & 1])

    def wait_write(c):
        pltpu.make_async_copy(out_v.at[c & 1], out_slice(c),
                              sem_w.at[c & 1]).wait()

    start_in(0)
    start_in(1)
    for c in range(NCHUNK):
        wait_in(c)
        start_gathers(c)
        if c >= 1:
            wait_gathers(c - 1)
            if c >= 3:
                wait_write(c - 3)
            assemble(c - 1)
            start_write(c - 1)
        if c + 2 < NCHUNK:
            start_in(c + 2)

    wait_gathers(NCHUNK - 1)
    wait_write(NCHUNK - 3)
    assemble(NCHUNK - 1)
    start_write(NCHUNK - 1)
    wait_write(NCHUNK - 2)
    wait_write(NCHUNK - 1)


@jax.jit
def kernel(continuous_data, categorical_data, tables):
    catt = categorical_data.astype(jnp.int32).T
    cont = continuous_data.reshape(BATCH * CONT_DIM)
    mesh = plsc.VectorSubcoreMesh(core_axis_name="c", subcore_axis_name="s")
    run = pl.kernel(
        _body,
        out_type=jax.ShapeDtypeStruct((BATCH * OUT_DIM,), jnp.float32),
        mesh=mesh,
        scratch_types=[
            pltpu.VMEM((N_FIELDS, ROWS_W), jnp.int32),
            pltpu.VMEM((3, CR * CONT_DIM + LANES,), jnp.float32),
            pltpu.VMEM((2, N_FIELDS, CR, EMBED_DIM), jnp.float32),
            pltpu.VMEM((2, CR * OUT_DIM,), jnp.float32),
            pltpu.SemaphoreType.DMA((3,)),
            pltpu.SemaphoreType.DMA((3,)),
            pltpu.SemaphoreType.DMA((2,)),
            pltpu.SemaphoreType.DMA((2,)),
        ],
        compiler_params=pltpu.CompilerParams(use_tc_tiling_on_sc=False),
    )
    out = run(cont, catt, tables)
    return out.reshape(BATCH, OUT_DIM)
